# baseline (speedup 1.0000x reference)
import functools

import jax
import jax.numpy as jnp
from jax import lax
from jax.experimental import pallas as pl
from jax.experimental.pallas import tpu as pltpu
from jax.experimental.pallas import tpu_sc as plsc

NC = 2
NS = 16
NW = NC * NS
LANES = 16
RW = 128
KR = 4
G = 16
DEPTH = 2


def _make_kernel(B, V, D):
    per_w = B // NW
    NG = per_w // G
    SUPER = NG // DEPTH
    n_full = D // LANES
    rem = D - n_full * LANES
    tail_base = D - LANES
    VR = (V * D) // RW
    assert V * D == VR * RW and (RW - 1) + D <= KR * RW

    mesh = plsc.VectorSubcoreMesh(core_axis_name="c", subcore_axis_name="s")

    @functools.partial(
        pl.kernel,
        mesh=mesh,
        compiler_params=pltpu.CompilerParams(
            use_tc_tiling_on_sc=True, needs_layout_passes=False),
        out_type=jax.ShapeDtypeStruct((B,), jnp.float32),
        scratch_types=[
            pltpu.VMEM((per_w, 2), jnp.int32),
            pltpu.VMEM((KR * per_w,), jnp.int32),
            pltpu.VMEM((KR * per_w,), jnp.int32),
            pltpu.VMEM((per_w,), jnp.int32),
            pltpu.VMEM((per_w,), jnp.int32),
            pltpu.VMEM((DEPTH, KR * G, RW), jnp.float32),
            pltpu.VMEM((DEPTH, KR * G, RW), jnp.float32),
            pltpu.VMEM((per_w,), jnp.float32),
            pltpu.SemaphoreType.DMA,
            pltpu.SemaphoreType.DMA,
            pltpu.SemaphoreType.DMA,
            pltpu.SemaphoreType.DMA,
        ],
    )
    def k(x_hbm, ine_hbm, oute_hbm, out_hbm,
          xv, ixi, ixo, svi, svo, rin, rout, res_v, si0, so0, si1, so1):
        wid = lax.axis_index("s") * NC + lax.axis_index("c")
        base = wid * per_w
        lane = lax.iota(jnp.int32, LANES)
        zero16 = lane * 0
        one16 = zero16 + 1
        tail_mask = lane >= (LANES - rem)
        perms = [lane ^ kk for kk in (8, 4, 2, 1)]
        sems = [(si0, so0), (si1, so1)]

        pltpu.sync_copy(x_hbm.at[pl.ds(base, per_w), :], xv)

        def build(cc, carry):
            q = cc * LANES + lane
            for col, ix, sv in ((zero16, ixi, svi), (one16, ixo, svo)):
                j = plsc.load_gather(xv, [q, col])
                flat = j * D
                r0 = flat // RW
                for kk in range(KR):
                    plsc.store_scatter(
                        ix, [q * KR + kk],
                        jnp.minimum(r0 + kk, VR - 1))
                plsc.store_scatter(sv, [q], flat - r0 * RW)
            return carry

        lax.fori_loop(0, per_w // LANES, build, 0)

        def fire(g, slot, sin, sout):
            pltpu.async_copy(
                ine_hbm.at[ixi.at[pl.ds(g * KR * G, KR * G)]],
                rin.at[slot], sin)
            pltpu.async_copy(
                oute_hbm.at[ixo.at[pl.ds(g * KR * G, KR * G)]],
                rout.at[slot], sout)

        def wait_slot(slot, sin, sout):
            pltpu.make_async_copy(
                ine_hbm.at[pl.ds(0, KR * G), :], rin.at[slot], sin).wait()
            pltpu.make_async_copy(
                oute_hbm.at[pl.ds(0, KR * G), :], rout.at[slot], sout).wait()

        def hsum_all(v):
            for p in perms:
                v = v + jnp.take_along_axis(
                    v, p, axis=0, mode="promise_in_bounds")
            return v

        def compute(g, slot):
            def half(h):
                def item(t, resvec):
                    tl = h * LANES + t
                    q16 = g * G + tl + zero16
                    wi = plsc.load_gather(svi, [q16]) + lane
                    wo = plsc.load_gather(svo, [q16]) + lane

                    def chunk(off, acc):
                        a, b = None, None
                        for w, buf in ((wi, rin), (wo, rout)):
                            wc = w + off
                            rr = lax.shift_right_logical(wc, 7)
                            v = plsc.load_gather(
                                buf.at[slot],
                                [KR * tl + rr, wc - rr * RW])
                            if a is None:
                                a = v
                            else:
                                b = v
                        return acc + a * b if acc is not None else a * b

                    acc = chunk(0, None)
                    for c in range(1, n_full):
                        acc = chunk(c * LANES, acc)
                    tail = chunk(tail_base, None)
                    acc += jnp.where(tail_mask, tail, jnp.float32(0.0))
                    return jnp.where(lane == t, hsum_all(acc), resvec)

                resvec = lax.fori_loop(
                    0, LANES, item, jnp.zeros((LANES,), jnp.float32))
                res_v[pl.ds(g * G + h * LANES, LANES)] = resvec

            for h in range(G // LANES):
                half(h)

        for s in range(DEPTH):
            fire(s, s, *sems[s])

        def super_body(kk, carry):
            g0 = kk * DEPTH
            for s in range(DEPTH):
                wait_slot(s, *sems[s])
                compute(g0 + s, s)
                fire(g0 + s + DEPTH, s, *sems[s])
            return carry

        lax.fori_loop(0, SUPER - 1, super_body, 0)

        for s in range(DEPTH):
            g = (SUPER - 1) * DEPTH + s
            wait_slot(s, *sems[s])
            compute(g, s)

        pltpu.sync_copy(res_v, out_hbm.at[pl.ds(base, per_w)])

    return k


@jax.jit
def kernel(x, input_embedding, output_embedding):
    B = x.shape[0]
    V, D = input_embedding.shape
    k = _make_kernel(B, V, D)
    return k(x, input_embedding.reshape((V * D) // RW, RW),
             output_embedding.reshape((V * D) // RW, RW))


# speedup vs baseline: 4.5274x; 4.5274x over previous
import functools

import jax
import jax.numpy as jnp
from jax import lax
from jax.experimental import pallas as pl
from jax.experimental.pallas import tpu as pltpu
from jax.experimental.pallas import tpu_sc as plsc

NC = 2
NS = 16
NW = NC * NS
LANES = 16
DEPTH = 2


def _make_kernel(B, V, D):
    per_w = B // NW
    NG = per_w // LANES
    SUPER = NG // DEPTH
    n_full = D // LANES
    rem = D - n_full * LANES
    tail_base = D - LANES

    mesh = plsc.VectorSubcoreMesh(core_axis_name="c", subcore_axis_name="s")

    @functools.partial(
        pl.kernel,
        mesh=mesh,
        compiler_params=pltpu.CompilerParams(
            use_tc_tiling_on_sc=True, needs_layout_passes=False),
        out_type=jax.ShapeDtypeStruct((B,), jnp.float32),
        scratch_types=[
            pltpu.VMEM((per_w, 2), jnp.int32),
            pltpu.VMEM((DEPTH, LANES, D), jnp.float32),
            pltpu.VMEM((DEPTH, LANES, D), jnp.float32),
            pltpu.VMEM((per_w,), jnp.float32),
            pltpu.SemaphoreType.DMA,
            pltpu.SemaphoreType.DMA,
            pltpu.SemaphoreType.DMA,
            pltpu.SemaphoreType.DMA,
        ],
    )
    def k(x_hbm, ine_hbm, oute_hbm, out_hbm,
          xv, rin, rout, res_v, si0, so0, si1, so1):
        wid = lax.axis_index("s") * NC + lax.axis_index("c")
        base = wid * per_w
        lane = lax.iota(jnp.int32, LANES)
        zero16 = lane * 0
        one16 = zero16 + 1
        tail_mask = lane >= (LANES - rem)
        perms = [lane ^ kk for kk in (8, 4, 2, 1)]
        sems = [(si0, so0), (si1, so1)]

        pltpu.sync_copy(x_hbm.at[pl.ds(base, per_w), :], xv)

        def fire(g, slot, sin, sout):
            rows = g * LANES + lane
            iv0 = plsc.load_gather(xv, [rows, zero16])
            iv1 = plsc.load_gather(xv, [rows, one16])
            for t in range(LANES):
                pltpu.async_copy(
                    ine_hbm.at[pl.ds(iv0[t], 1), :],
                    rin.at[slot, pl.ds(t, 1), :], sin)
                pltpu.async_copy(
                    oute_hbm.at[pl.ds(iv1[t], 1), :],
                    rout.at[slot, pl.ds(t, 1), :], sout)

        def wait_slot(slot, sin, sout):
            pltpu.make_async_copy(
                ine_hbm.at[pl.ds(0, LANES), :], rin.at[slot], sin).wait()
            pltpu.make_async_copy(
                oute_hbm.at[pl.ds(0, LANES), :], rout.at[slot], sout).wait()

        def hsum_all(v):
            for p in perms:
                v = v + jnp.take_along_axis(
                    v, p, axis=0, mode="promise_in_bounds")
            return v

        def compute(g, slot):
            def item(t, resvec):
                acc = (rin[slot, t, pl.ds(0, LANES)]
                       * rout[slot, t, pl.ds(0, LANES)])
                for c in range(1, n_full):
                    acc += (rin[slot, t, pl.ds(c * LANES, LANES)]
                            * rout[slot, t, pl.ds(c * LANES, LANES)])
                tail = (rin[slot, t, pl.ds(tail_base, LANES)]
                        * rout[slot, t, pl.ds(tail_base, LANES)])
                acc += jnp.where(tail_mask, tail, jnp.float32(0.0))
                return jnp.where(lane == t, hsum_all(acc), resvec)

            resvec = lax.fori_loop(
                0, LANES, item, jnp.zeros((LANES,), jnp.float32))
            res_v[pl.ds(g * LANES, LANES)] = resvec

        for s in range(DEPTH):
            fire(s, s, *sems[s])

        def super_body(kk, carry):
            g0 = kk * DEPTH
            for s in range(DEPTH):
                wait_slot(s, *sems[s])
                compute(g0 + s, s)
                fire(g0 + s + DEPTH, s, *sems[s])
            return carry

        lax.fori_loop(0, SUPER - 1, super_body, 0)

        for s in range(DEPTH):
            g = (SUPER - 1) * DEPTH + s
            wait_slot(s, *sems[s])
            compute(g, s)

        pltpu.sync_copy(res_v, out_hbm.at[pl.ds(base, per_w)])

    return k


@jax.jit
def kernel(x, input_embedding, output_embedding):
    B = x.shape[0]
    V, D = input_embedding.shape
    k = _make_kernel(B, V, D)
    return k(x, input_embedding, output_embedding)
